# baseline (speedup 1.0000x reference)
import functools

import jax
import jax.numpy as jnp
from jax import lax
from jax.experimental import pallas as pl
from jax.experimental.pallas import tpu as pltpu
from jax.experimental.pallas import tpu_sc as plsc

B = 16384
D = 64
NC = 2
NS = 16
NW = NC * NS
B_PER_W = B // NW
CHUNK = 128
NCHUNK = B_PER_W // CHUNK


def _sc_gather_body(uidx_hbm, iidx_hbm, tu_mlp, ti_mlp, tu_mf, ti_mf,
                    ou_mlp, oi_mlp, ou_mf, oi_mf,
                    uidx_v, iidx_v, bu_mlp, bi_mlp, bu_mf, bi_mf,
                    s0, s1, s2, s3):
    wid = lax.axis_index("s") * NC + lax.axis_index("c")
    base = wid * B_PER_W
    for c in range(NCHUNK):
        off = base + c * CHUNK
        pltpu.sync_copy(uidx_hbm.at[pl.ds(off, CHUNK)], uidx_v)
        pltpu.sync_copy(iidx_hbm.at[pl.ds(off, CHUNK)], iidx_v)
        d0 = pltpu.async_copy(tu_mlp.at[uidx_v], bu_mlp, s0)
        d1 = pltpu.async_copy(ti_mlp.at[iidx_v], bi_mlp, s1)
        d2 = pltpu.async_copy(tu_mf.at[uidx_v], bu_mf, s2)
        d3 = pltpu.async_copy(ti_mf.at[iidx_v], bi_mf, s3)
        d0.wait()
        pltpu.sync_copy(bu_mlp, ou_mlp.at[pl.ds(off, CHUNK)])
        d1.wait()
        pltpu.sync_copy(bi_mlp, oi_mlp.at[pl.ds(off, CHUNK)])
        d2.wait()
        pltpu.sync_copy(bu_mf, ou_mf.at[pl.ds(off, CHUNK)])
        d3.wait()
        pltpu.sync_copy(bi_mf, oi_mf.at[pl.ds(off, CHUNK)])


_sc_gather = functools.partial(
    pl.kernel,
    out_type=[jax.ShapeDtypeStruct((B, D), jnp.float32)] * 4,
    mesh=plsc.VectorSubcoreMesh(core_axis_name="c", subcore_axis_name="s",
                                num_cores=NC, num_subcores=NS),
    scratch_types=[
        pltpu.VMEM((CHUNK,), jnp.int32),
        pltpu.VMEM((CHUNK,), jnp.int32),
        pltpu.VMEM((CHUNK, D), jnp.float32),
        pltpu.VMEM((CHUNK, D), jnp.float32),
        pltpu.VMEM((CHUNK, D), jnp.float32),
        pltpu.VMEM((CHUNK, D), jnp.float32),
        pltpu.SemaphoreType.DMA,
        pltpu.SemaphoreType.DMA,
        pltpu.SemaphoreType.DMA,
        pltpu.SemaphoreType.DMA,
    ],
    compiler_params=pltpu.CompilerParams(use_tc_tiling_on_sc=False),
)(_sc_gather_body)


def _mlp_body(u_mlp, i_mlp, u_mf, i_mf, w1a, w1b, b1, w2, b2, wa1, wa2, ba,
              out_ref):
    x = jnp.dot(u_mlp[...], w1a[...], preferred_element_type=jnp.float32)
    x = x + jnp.dot(i_mlp[...], w1b[...], preferred_element_type=jnp.float32)
    x = jnp.maximum(x + b1[...], 0.0)
    x = jnp.dot(x, w2[...], preferred_element_type=jnp.float32) + b2[...]
    x = jnp.maximum(x, 0.0)
    mf = u_mf[...] * i_mf[...]
    z = jnp.dot(x, wa1[...], preferred_element_type=jnp.float32)
    z = z + jnp.dot(mf, wa2[...], preferred_element_type=jnp.float32)
    z = z + ba[0, 0]
    out_ref[...] = (1.0 / (1.0 + jnp.exp(-z)))[:, 0]


def _run_mlp(u_mlp, i_mlp, u_mf, i_mf, W1, b1, W2, b2, Wa, ba):
    R = 2048
    grid = (B // R,)
    row_spec = pl.BlockSpec((R, D), lambda i: (i, 0))
    full = lambda shape: pl.BlockSpec(shape, lambda i: (0,) * len(shape))
    return pl.pallas_call(
        _mlp_body,
        grid=grid,
        in_specs=[row_spec, row_spec, row_spec, row_spec,
                  full((64, 64)), full((64, 64)), full((1, 64)),
                  full((64, 32)), full((1, 32)),
                  full((32, 1)), full((64, 1)), full((1, 1))],
        out_specs=pl.BlockSpec((R,), lambda i: (i,)),
        out_shape=jax.ShapeDtypeStruct((B,), jnp.float32),
    )(u_mlp, i_mlp, u_mf, i_mf,
      W1[:64], W1[64:], b1.reshape(1, 64),
      W2, b2.reshape(1, 32),
      Wa[:32], Wa[32:], ba.reshape(1, 1))


def kernel(user_indices, item_indices, emb_user_mlp, emb_item_mlp,
           emb_user_mf, emb_item_mf, W1, b1, W2, b2, Wa, ba):
    u_mlp, i_mlp, u_mf, i_mf = _sc_gather(
        user_indices, item_indices,
        emb_user_mlp, emb_item_mlp, emb_user_mf, emb_item_mf)
    return _run_mlp(u_mlp, i_mlp, u_mf, i_mf, W1, b1, W2, b2, Wa, ba)
